# baseline (speedup 1.0000x reference)
import functools

import jax
import jax.numpy as jnp
from jax import lax
from jax.experimental import pallas as pl
from jax.experimental.pallas import tpu as pltpu
from jax.experimental.pallas import tpu_sc as plsc

VOCAB = 1000000
EMB = 32
BATCH = 16384
FIELDS = 26
TOTAL = BATCH * FIELDS

PACK = 4
BLK = PACK * EMB
NUM_CORES = 2
NUM_SUBCORES = 16
NUM_WORKERS = NUM_CORES * NUM_SUBCORES
ROWS_PER_WORKER = BATCH // NUM_WORKERS
PER_WORKER = ROWS_PER_WORKER * FIELDS

CHUNK = 256
NUM_CHUNKS = PER_WORKER // CHUNK
GROUPS = CHUNK // 16

FB_ROWS = 8
FB_CHUNKS = ROWS_PER_WORKER // FB_ROWS

_MESH = dict(core_axis_name="c", subcore_axis_name="s")


def _ids_format(env_ids):
  @functools.partial(
      pl.kernel,
      mesh=plsc.VectorSubcoreMesh(**_MESH),
      compiler_params=pltpu.CompilerParams(needs_layout_passes=False),
      out_type=jax.ShapeDtypeStruct((TOTAL,), jnp.int32),
      scratch_types=[
          pltpu.VMEM((ROWS_PER_WORKER, FIELDS), jnp.int32),
          pltpu.VMEM((PER_WORKER,), jnp.int32),
      ],
  )
  def k(ids_hbm, out_hbm, ids_v, flat_v):
    wid = lax.axis_index("s") * NUM_CORES + lax.axis_index("c")
    brow = wid * ROWS_PER_WORKER
    pltpu.sync_copy(ids_hbm.at[pl.ds(brow, ROWS_PER_WORKER)], ids_v)
    lane = lax.iota(jnp.int32, 16)
    hi_col = jnp.minimum(lane + 16, FIELDS - 1)
    hi_off = jnp.minimum(lane, 9) + 16

    def row_body(r, carry):
      rb = jnp.full((16,), r, jnp.int32)
      lo = ids_v.at[r][pl.ds(0, 16)]
      hi = plsc.load_gather(ids_v, [rb, hi_col])
      plsc.store_scatter(flat_v, [r * FIELDS + lane], lo)
      plsc.store_scatter(flat_v, [r * FIELDS + hi_off], hi)
      return carry

    lax.fori_loop(0, ROWS_PER_WORKER, row_body, 0)
    pltpu.sync_copy(flat_v, out_hbm.at[pl.ds(brow * FIELDS, PER_WORKER)])

  return k(env_ids)


def _gather(idx_flat, table_blocks):
  @functools.partial(
      pl.kernel,
      mesh=plsc.VectorSubcoreMesh(**_MESH),
      compiler_params=pltpu.CompilerParams(
          use_tc_tiling_on_sc=False, needs_layout_passes=False
      ),
      out_type=jax.ShapeDtypeStruct((TOTAL * EMB,), jnp.float32),
      scratch_types=[
          pltpu.VMEM((PER_WORKER,), jnp.int32),
          pltpu.VMEM((CHUNK,), jnp.int32),
          pltpu.VMEM((CHUNK,), jnp.int32),
          pltpu.VMEM((CHUNK, BLK), jnp.float32),
          pltpu.VMEM((CHUNK, BLK), jnp.float32),
          pltpu.VMEM((CHUNK * EMB,), jnp.float32),
          pltpu.VMEM((CHUNK * EMB,), jnp.float32),
          pltpu.SemaphoreType.DMA,
          pltpu.SemaphoreType.DMA,
          pltpu.SemaphoreType.DMA,
          pltpu.SemaphoreType.DMA,
      ],
  )
  def k(idx_hbm, table_hbm, out_hbm, idx_v, bidx0, bidx1, rows0, rows1,
        sel0, sel1, g0, g1, w0, w1):
    bidx_r = [bidx0, bidx1]
    rows_r = [rows0, rows1]
    sel_r = [sel0, sel1]
    wid = lax.axis_index("s") * NUM_CORES + lax.axis_index("c")
    base = wid * PER_WORKER
    gsem = [g0, g1]
    wsem = [w0, w1]
    lane = lax.iota(jnp.int32, 16)
    three = jnp.full((16,), 3, jnp.int32)

    pltpu.sync_copy(idx_hbm.at[pl.ds(base, PER_WORKER)], idx_v)

    def gather(c, b):
      def blk_body(g, carry):
        v = idx_v[pl.ds(c * CHUNK + g * 16, 16)]
        bidx_r[b][pl.ds(g * 16, 16)] = lax.shift_right_logical(v, 2)
        return carry

      lax.fori_loop(0, GROUPS, blk_body, 0)
      pltpu.async_copy(table_hbm.at[bidx_r[b]], rows_r[b], gsem[b])

    def wait_gather(b):
      pltpu.make_async_copy(
          table_hbm.at[pl.ds(0, CHUNK)], rows_r[b], gsem[b]
      ).wait()

    def select(c, b):
      def sel_body(g, carry):
        v = idx_v[pl.ds(c * CHUNK + g * 16, 16)]
        q32 = (v & three) * EMB
        row = lane + g * 16
        dstb = row * EMB
        for d in range(EMB):
          jd = (lane + d) & (EMB - 1)
          vals = plsc.load_gather(rows_r[b], [row, q32 + jd])
          plsc.store_scatter(sel_r[b], [dstb + jd], vals)
        return carry

      lax.fori_loop(0, GROUPS, sel_body, 0)

    def write(c, b):
      pltpu.async_copy(
          sel_r[b],
          out_hbm.at[pl.ds((base + c * CHUNK) * EMB, CHUNK * EMB)],
          wsem[b],
      )

    def wait_write(b):
      pltpu.make_async_copy(
          out_hbm.at[pl.ds(0, CHUNK * EMB)], sel_r[b], wsem[b]
      ).wait()

    gather(0, 0)
    gather(1, 1)

    def step_body(s, carry):
      for b in range(2):
        c = s * 2 + b
        wait_gather(b)
        select(c, b)
        write(c, b)
        wait_write(b)
        nxt = c + 2

        @pl.when(nxt < NUM_CHUNKS)
        def _():
          gather(nxt, b)

      return carry

    lax.fori_loop(0, NUM_CHUNKS // 2, step_body, 0)

  return k(idx_flat, table_blocks)


def _out_format(flat):
  @functools.partial(
      pl.kernel,
      mesh=plsc.VectorSubcoreMesh(**_MESH),
      compiler_params=pltpu.CompilerParams(needs_layout_passes=False),
      out_type=jax.ShapeDtypeStruct((BATCH, FIELDS, EMB), jnp.float32),
      scratch_types=[
          pltpu.VMEM((FB_ROWS * FIELDS * EMB,), jnp.float32),
          pltpu.VMEM((FB_ROWS * FIELDS * EMB,), jnp.float32),
          pltpu.VMEM((FB_ROWS, FIELDS, EMB), jnp.float32),
          pltpu.VMEM((FB_ROWS, FIELDS, EMB), jnp.float32),
          pltpu.SemaphoreType.DMA,
          pltpu.SemaphoreType.DMA,
          pltpu.SemaphoreType.DMA,
          pltpu.SemaphoreType.DMA,
      ],
  )
  def k(flat_hbm, out_hbm, fin0, fin1, pout0, pout1, g0, g1, w0, w1):
    fin_r = [fin0, fin1]
    pout_r = [pout0, pout1]
    wid = lax.axis_index("s") * NUM_CORES + lax.axis_index("c")
    brow = wid * ROWS_PER_WORKER
    gsem = [g0, g1]
    wsem = [w0, w1]
    n = FB_ROWS * FIELDS * EMB

    def read(c):
      b = c % 2
      return pltpu.async_copy(
          flat_hbm.at[pl.ds((brow + c * FB_ROWS) * FIELDS * EMB, n)],
          fin_r[b],
          gsem[b],
      )

    def assemble(c):
      b = c % 2

      def row_body(r, carry):
        bb = r // FIELDS
        f = r - bb * FIELDS
        src = fin_r[b]
        dst = pout_r[b].at[bb].at[f]
        dst[pl.ds(0, 16)] = src[pl.ds(r * EMB, 16)]
        dst[pl.ds(16, 16)] = src[pl.ds(r * EMB + 16, 16)]
        return carry

      lax.fori_loop(0, FB_ROWS * FIELDS, row_body, 0)

    def write(c):
      b = c % 2
      return pltpu.async_copy(
          pout_r[b],
          out_hbm.at[pl.ds(brow + c * FB_ROWS, FB_ROWS)],
          wsem[b],
      )

    hg = {}
    hw = {}
    hg[0] = read(0)
    for c in range(FB_CHUNKS):
      if c + 1 < FB_CHUNKS:
        if c + 1 >= 2:
          hw[c - 1].wait()
        hg[c + 1] = read(c + 1)
      hg[c].wait()
      assemble(c)
      hw[c] = write(c)
    hw[FB_CHUNKS - 2].wait()
    hw[FB_CHUNKS - 1].wait()

  return k(flat)


def kernel(env_ids, table):
  idx_flat = _ids_format(env_ids.astype(jnp.int32))
  table_blocks = table.reshape(VOCAB // PACK, BLK)
  flat = _gather(idx_flat, table_blocks)
  return _out_format(flat)


# speedup vs baseline: 1.0620x; 1.0620x over previous
import functools

import jax
import jax.numpy as jnp
from jax import lax
from jax.experimental import pallas as pl
from jax.experimental.pallas import tpu as pltpu
from jax.experimental.pallas import tpu_sc as plsc

VOCAB = 1000000
EMB = 32
BATCH = 16384
FIELDS = 26
TOTAL = BATCH * FIELDS

PACK = 4
BLK = PACK * EMB
NUM_CORES = 2
NUM_SUBCORES = 16
NUM_WORKERS = NUM_CORES * NUM_SUBCORES
ROWS_PER_WORKER = BATCH // NUM_WORKERS
PER_WORKER = ROWS_PER_WORKER * FIELDS

CB = 8
CHUNK = CB * FIELDS
GROUPS = CHUNK // 16
NUM_CHUNKS = ROWS_PER_WORKER // CB

_MESH = dict(core_axis_name="c", subcore_axis_name="s")


def _ids_format(env_ids):
  @functools.partial(
      pl.kernel,
      mesh=plsc.VectorSubcoreMesh(**_MESH),
      compiler_params=pltpu.CompilerParams(needs_layout_passes=False),
      out_type=jax.ShapeDtypeStruct((TOTAL,), jnp.int32),
      scratch_types=[
          pltpu.VMEM((ROWS_PER_WORKER, FIELDS), jnp.int32),
          pltpu.VMEM((PER_WORKER,), jnp.int32),
      ],
  )
  def k(ids_hbm, out_hbm, ids_v, flat_v):
    wid = lax.axis_index("s") * NUM_CORES + lax.axis_index("c")
    brow = wid * ROWS_PER_WORKER
    pltpu.sync_copy(ids_hbm.at[pl.ds(brow, ROWS_PER_WORKER)], ids_v)
    lane = lax.iota(jnp.int32, 16)
    hi_col = jnp.minimum(lane + 16, FIELDS - 1)
    hi_off = jnp.minimum(lane, 9) + 16

    def row_body(r, carry):
      rb = jnp.full((16,), r, jnp.int32)
      lo = ids_v.at[r][pl.ds(0, 16)]
      hi = plsc.load_gather(ids_v, [rb, hi_col])
      plsc.store_scatter(flat_v, [r * FIELDS + lane], lo)
      plsc.store_scatter(flat_v, [r * FIELDS + hi_off], hi)
      return carry

    lax.fori_loop(0, ROWS_PER_WORKER, row_body, 0)
    pltpu.sync_copy(flat_v, out_hbm.at[pl.ds(brow * FIELDS, PER_WORKER)])

  return k(env_ids)


def _gather(idx_flat, table_blocks):
  @functools.partial(
      pl.kernel,
      mesh=plsc.VectorSubcoreMesh(**_MESH),
      compiler_params=pltpu.CompilerParams(needs_layout_passes=False),
      out_type=jax.ShapeDtypeStruct((BATCH, FIELDS, EMB), jnp.float32),
      scratch_types=[
          pltpu.VMEM((CHUNK,), jnp.int32),
          pltpu.VMEM((CHUNK,), jnp.int32),
          pltpu.VMEM((CHUNK,), jnp.int32),
          pltpu.VMEM((CHUNK,), jnp.int32),
          pltpu.VMEM((CHUNK, BLK), jnp.float32),
          pltpu.VMEM((CHUNK, BLK), jnp.float32),
          pltpu.VMEM((CHUNK * EMB,), jnp.float32),
          pltpu.VMEM((CHUNK * EMB,), jnp.float32),
          pltpu.VMEM((CB, FIELDS, EMB), jnp.float32),
          pltpu.SemaphoreType.DMA,
          pltpu.SemaphoreType.DMA,
          pltpu.SemaphoreType.DMA,
          pltpu.SemaphoreType.DMA,
      ],
  )
  def k(idx_hbm, table_hbm, out_hbm, idx0, idx1, bidx0, bidx1,
        rows0, rows1, sel0, sel1, pout, g0, g1, w0, w1):
    idx_r = [idx0, idx1]
    bidx_r = [bidx0, bidx1]
    rows_r = [rows0, rows1]
    sel_r = [sel0, sel1]
    gsem = [g0, g1]
    wsem = [w0, w1]
    wid = lax.axis_index("s") * NUM_CORES + lax.axis_index("c")
    base = wid * PER_WORKER
    brow = wid * ROWS_PER_WORKER
    lane = lax.iota(jnp.int32, 16)
    three = jnp.full((16,), 3, jnp.int32)
    fields = jnp.full((16,), FIELDS, jnp.int32)

    def gather(c, b):
      pltpu.sync_copy(idx_hbm.at[pl.ds(base + c * CHUNK, CHUNK)], idx_r[b])

      def blk_body(g, carry):
        v = idx_r[b][pl.ds(g * 16, 16)]
        bidx_r[b][pl.ds(g * 16, 16)] = lax.shift_right_logical(v, 2)
        return carry

      lax.fori_loop(0, GROUPS, blk_body, 0)
      pltpu.async_copy(table_hbm.at[bidx_r[b]], rows_r[b], gsem[b])

    def wait_gather(b):
      pltpu.make_async_copy(
          table_hbm.at[pl.ds(0, CHUNK)], rows_r[b], gsem[b]
      ).wait()

    def select(c, b):
      def sel_body(g, carry):
        v = idx_r[b][pl.ds(g * 16, 16)]
        q32 = (v & three) * EMB
        rl = lane + g * 16
        dstb = rl * EMB
        for d in range(EMB):
          jd = (lane + d) & (EMB - 1)
          vals = plsc.load_gather(rows_r[b], [rl, q32 + jd])
          plsc.store_scatter(sel_r[b], [dstb + jd], vals)
        return carry

      lax.fori_loop(0, GROUPS, sel_body, 0)

    def assemble(b):
      def row_body(r, carry):
        bb = r // FIELDS
        ff = r - bb * FIELDS
        dst = pout.at[bb].at[ff]
        dst[pl.ds(0, 16)] = sel_r[b][pl.ds(r * EMB, 16)]
        dst[pl.ds(16, 16)] = sel_r[b][pl.ds(r * EMB + 16, 16)]
        return carry

      lax.fori_loop(0, CHUNK, row_body, 0)

    def write(c):
      pltpu.async_copy(
          pout,
          out_hbm.at[pl.ds(brow + c * CB, CB)],
          w0,
      )

    def wait_write():
      pltpu.make_async_copy(
          out_hbm.at[pl.ds(0, CB)], pout, w0
      ).wait()

    gather(0, 0)
    gather(1, 1)

    def step_body(s, carry):
      for b in range(2):
        c = s * 2 + b
        wait_gather(b)
        select(c, b)

        @pl.when(c > 0)
        def _():
          wait_write()

        assemble(b)
        write(c)
        nxt = c + 2

        @pl.when(nxt < NUM_CHUNKS)
        def _():
          gather(nxt, b)

      return carry

    lax.fori_loop(0, NUM_CHUNKS // 2, step_body, 0)
    wait_write()

  return k(idx_flat, table_blocks)


def kernel(env_ids, table):
  idx_flat = _ids_format(env_ids.astype(jnp.int32))
  table_blocks = table.reshape(VOCAB // PACK, BLK)
  return _gather(idx_flat, table_blocks)


# speedup vs baseline: 1.2677x; 1.1937x over previous
import functools

import jax
import jax.numpy as jnp
from jax import lax
from jax.experimental import pallas as pl
from jax.experimental.pallas import tpu as pltpu
from jax.experimental.pallas import tpu_sc as plsc

VOCAB = 1000000
EMB = 32
BATCH = 16384
FIELDS = 26
FPAD = 32

NUM_CORES = 2
NUM_SUBCORES = 16
NUM_WORKERS = NUM_CORES * NUM_SUBCORES
ROWS_PER_WORKER = BATCH // NUM_WORKERS
CHUNK_ROWS = 32
NUM_CHUNKS = ROWS_PER_WORKER // CHUNK_ROWS

_MESH = dict(core_axis_name="c", subcore_axis_name="s")


def _ids_format(env_ids):
  @functools.partial(
      pl.kernel,
      mesh=plsc.VectorSubcoreMesh(**_MESH),
      compiler_params=pltpu.CompilerParams(needs_layout_passes=False),
      out_type=jax.ShapeDtypeStruct((BATCH * FPAD,), jnp.int32),
      scratch_types=[
          pltpu.VMEM((ROWS_PER_WORKER, FIELDS), jnp.int32),
          pltpu.VMEM((ROWS_PER_WORKER * FPAD,), jnp.int32),
      ],
  )
  def k(ids_hbm, out_hbm, ids_v, flat_v):
    wid = lax.axis_index("s") * NUM_CORES + lax.axis_index("c")
    brow = wid * ROWS_PER_WORKER
    pltpu.sync_copy(ids_hbm.at[pl.ds(brow, ROWS_PER_WORKER)], ids_v)
    lane = lax.iota(jnp.int32, 16)
    hi_col = jnp.minimum(lane + 16, FIELDS - 1)

    def row_body(r, carry):
      lo = ids_v.at[r][pl.ds(0, 16)]
      rb = jnp.full((16,), r, jnp.int32)
      hi = plsc.load_gather(ids_v, [rb, hi_col])
      flat_v[pl.ds(r * FPAD, 16)] = lo
      flat_v[pl.ds(r * FPAD + 16, 16)] = hi
      return carry

    lax.fori_loop(0, ROWS_PER_WORKER, row_body, 0)
    pltpu.sync_copy(
        flat_v, out_hbm.at[pl.ds(brow * FPAD, ROWS_PER_WORKER * FPAD)]
    )

  return k(env_ids)


def _embedding_gather(idx_flat, table):
  @functools.partial(
      pl.kernel,
      mesh=plsc.VectorSubcoreMesh(**_MESH),
      compiler_params=pltpu.CompilerParams(use_tc_tiling_on_sc=False),
      out_type=jax.ShapeDtypeStruct((BATCH, FIELDS, EMB), jnp.float32),
      scratch_types=[
          pltpu.VMEM((ROWS_PER_WORKER * FPAD,), jnp.int32),
          pltpu.VMEM((2, CHUNK_ROWS, FIELDS, EMB), jnp.float32),
          pltpu.SemaphoreType.DMA,
          pltpu.SemaphoreType.DMA,
          pltpu.SemaphoreType.DMA,
          pltpu.SemaphoreType.DMA,
      ],
  )
  def k(idx_hbm, table_hbm, out_hbm, idx_v, rows_v, g0, g1, w0, w1):
    wid = lax.axis_index("s") * NUM_CORES + lax.axis_index("c")
    brow = wid * ROWS_PER_WORKER
    gsem = [g0, g1]
    wsem = [w0, w1]

    pltpu.sync_copy(
        idx_hbm.at[pl.ds(brow * FPAD, ROWS_PER_WORKER * FPAD)], idx_v
    )

    def gather(c):
      b = c % 2

      def row_body(i, carry):
        pltpu.async_copy(
            table_hbm.at[idx_v.at[pl.ds((c * CHUNK_ROWS + i) * FPAD, FIELDS)]],
            rows_v.at[b].at[i],
            gsem[b],
        )
        return carry

      lax.fori_loop(0, CHUNK_ROWS, row_body, 0)
      return pltpu.make_async_copy(
          out_hbm.at[pl.ds(0, CHUNK_ROWS)], rows_v.at[b], gsem[b]
      )

    def write(c):
      b = c % 2
      return pltpu.async_copy(
          rows_v.at[b],
          out_hbm.at[pl.ds(brow + c * CHUNK_ROWS, CHUNK_ROWS)],
          wsem[b],
      )

    hg = {}
    hw = {}
    hg[0] = gather(0)
    for c in range(NUM_CHUNKS):
      if c + 1 < NUM_CHUNKS:
        if c + 1 >= 2:
          hw[c - 1].wait()
        hg[c + 1] = gather(c + 1)
      hg[c].wait()
      hw[c] = write(c)
    hw[NUM_CHUNKS - 2].wait()
    hw[NUM_CHUNKS - 1].wait()

  return k(idx_flat, table)


def kernel(env_ids, table):
  idx_flat = _ids_format(env_ids.astype(jnp.int32))
  return _embedding_gather(idx_flat, table)
